# Initial kernel scaffold; baseline (speedup 1.0000x reference)
import jax
import jax.numpy as jnp
from jax.experimental import pallas as pl


def kernel(res_H, res_X, res_S, batch, ligand_pos, ligand_feat, ligand_mask, edit_residue_num, residue_mask, W_Q, W_K, W_V, W_O, ln_g, ln_b, sD_w1, sD_b1, sD_w2, sD_b2, Ti_w1, Ti_b1, Ti_w2, Ti_b2):
    raise NotImplementedError("write your pallas kernel here")


# speedup vs baseline: 1.6681x; 1.6681x over previous
import functools

import jax
import jax.numpy as jnp
import numpy as np
from jax.experimental import pallas as pl
from jax.experimental.pallas import tpu as pltpu

N_NODES = 256
N_CH = 14
HID = 128
ECH = 64
HEADS = 4
KNN = 8
CUTOFF = 10.0
K9 = KNN + 1
KQ = K9 * N_CH

_RAM = np.array([
    [1, 1, 1, 1, 1, 1, 1, 1, 1, 1, 1, 1, 1, 1],
    [1, 1, 1, 1, 1, 0, 0, 0, 0, 0, 0, 0, 0, 0],
    [1, 1, 1, 1, 1, 1, 1, 1, 1, 1, 1, 0, 0, 0],
    [1, 1, 1, 1, 1, 1, 1, 1, 0, 0, 0, 0, 0, 0],
    [1, 1, 1, 1, 1, 1, 1, 1, 0, 0, 0, 0, 0, 0],
    [1, 1, 1, 1, 1, 1, 0, 0, 0, 0, 0, 0, 0, 0],
    [1, 1, 1, 1, 1, 1, 1, 1, 1, 0, 0, 0, 0, 0],
    [1, 1, 1, 1, 1, 1, 1, 1, 1, 0, 0, 0, 0, 0],
    [1, 1, 1, 1, 0, 0, 0, 0, 0, 0, 0, 0, 0, 0],
    [1, 1, 1, 1, 1, 1, 1, 1, 1, 1, 0, 0, 0, 0],
    [1, 1, 1, 1, 1, 1, 1, 1, 0, 0, 0, 0, 0, 0],
    [1, 1, 1, 1, 1, 1, 1, 1, 0, 0, 0, 0, 0, 0],
    [1, 1, 1, 1, 1, 1, 1, 1, 1, 0, 0, 0, 0, 0],
    [1, 1, 1, 1, 1, 1, 1, 1, 0, 0, 0, 0, 0, 0],
    [1, 1, 1, 1, 1, 1, 1, 1, 1, 1, 1, 0, 0, 0],
    [1, 1, 1, 1, 1, 1, 1, 0, 0, 0, 0, 0, 0, 0],
    [1, 1, 1, 1, 1, 1, 0, 0, 0, 0, 0, 0, 0, 0],
    [1, 1, 1, 1, 1, 1, 1, 0, 0, 0, 0, 0, 0, 0],
    [1, 1, 1, 1, 1, 1, 1, 1, 1, 1, 1, 1, 1, 1],
    [1, 1, 1, 1, 1, 1, 1, 1, 1, 1, 1, 1, 0, 0],
    [1, 1, 1, 1, 1, 1, 1, 0, 0, 0, 0, 0, 0, 0]], dtype=np.float32)


def _prologue_body(h_ref, s_ref, wq_ref, wk_ref, wv_ref, ta_ref, tb_ref,
                   g_ref, b_ref, i21_ref, ram_ref,
                   hln_ref, q_ref, k_ref, v_ref, a_ref, bb_ref, am_ref):
    x = h_ref[:, :]
    mu = jnp.mean(x, axis=1, keepdims=True)
    var = jnp.mean((x - mu) ** 2, axis=1, keepdims=True)
    hln = (x - mu) / jnp.sqrt(var + 1e-5) * g_ref[:, :] + b_ref[:, :]
    hln_ref[:, :] = hln
    cd = (((1,), (1,)), ((), ()))
    q = jax.lax.dot_general(hln, wq_ref[:, :], cd,
                            preferred_element_type=jnp.float32)
    k = jax.lax.dot_general(hln, wk_ref[:, :], cd,
                            preferred_element_type=jnp.float32)
    v = jax.lax.dot_general(hln, wv_ref[:, :], cd,
                            preferred_element_type=jnp.float32)
    q_ref[:, :] = q
    k_ref[:, :] = k
    v_ref[:, :] = v
    a_ref[:, :] = jax.lax.dot_general(q, ta_ref[:, :], cd,
                                      preferred_element_type=jnp.float32)
    bb_ref[:, :] = jax.lax.dot_general(k, tb_ref[:, :], cd,
                                       preferred_element_type=jnp.float32)
    oh = (s_ref[:, :] == i21_ref[:, :]).astype(jnp.float32)
    am_ref[:, :] = jax.lax.dot_general(
        oh, ram_ref[:, :], (((1,), (0,)), ((), ())),
        preferred_element_type=jnp.float32)


def _knn_body(pc_ref, pt_ref, eye_ref, coli_ref, nbr_ref):
    big = jnp.float32(1e30)
    d2 = eye_ref[:, :]
    for c in range(3):
        diff = pc_ref[:, c:c + 1] - pt_ref[c:c + 1, :]
        d2 = d2 + diff * diff
    coli = coli_ref[:, :]
    for k in range(KNN):
        m = jnp.min(d2, axis=1, keepdims=True)
        idx = jnp.min(jnp.where(d2 <= m, coli, jnp.float32(N_NODES)),
                      axis=1, keepdims=True)
        nbr_ref[:, k:k + 1] = idx
        d2 = jnp.where(coli == idx, big, d2)


def _main_body(nbr_ref, k_ref, v_ref, b_ref, x_ref, am_ref,
               q_ref, a_ref, hln_ref, xin_ref,
               wg_ref, sdb1_ref, sdw2_ref, sdb2_ref, tib1_ref, tiw2_ref,
               tib2_ref, wo_ref, offs_ref, gc_ref, qoh_ref, bs_ref, sp_ref,
               qm_ref,
               ho_ref, xo_ref,
               kg, vg, bg, xg, amg, f_sc, l_sc):
    i = pl.program_id(0)
    cd11 = (((1,), (1,)), ((), ()))
    cd10 = (((1,), (0,)), ((), ()))
    f32 = jnp.float32

    for k in range(K9):
        j = nbr_ref[i, k]
        kg[k * N_CH:(k + 1) * N_CH, :] = k_ref[j]
        vg[k * N_CH:(k + 1) * N_CH, :] = v_ref[j]
        bg[k * N_CH:(k + 1) * N_CH, :] = b_ref[j]
        xg[k:k + 1, :] = x_ref[j]
        amg[k:k + 1, :] = am_ref[j]

    Kg = kg[:, :]
    Vg = vg[:, :]
    Bg = bg[:, :]
    Gc = gc_ref[:, :]
    QOH = qoh_ref[:, :]
    xi = x_ref[i]
    am_i_row = am_ref[i]
    am_i_col = am_i_row.T

    XREP = jax.lax.dot_general(Gc, xg[:, :], cd10,
                               preferred_element_type=f32)
    AMrep = jax.lax.dot_general(Gc, amg[:, :], cd10,
                                preferred_element_type=f32)
    am_j_col = jnp.sum(AMrep * QOH, axis=1, keepdims=True)
    am_j_row = am_j_col.T

    xj_cols = []
    d2kq = jnp.zeros((KQ, N_CH), f32)
    for c in range(3):
        xj_c = jnp.sum(XREP[:, c * N_CH:(c + 1) * N_CH] * QOH,
                       axis=1, keepdims=True)
        xj_cols.append(xj_c)
        dc = xj_c - xi[:, c * N_CH:(c + 1) * N_CH]
        d2kq = d2kq + dc * dc
    Rkq = jnp.sqrt(d2kq + 1e-12)

    Q_i = q_ref[0]
    Qh = jnp.concatenate([Q_i, Q_i, Q_i, Q_i], axis=0) * qm_ref[:, :]
    qk = jax.lax.dot_general(Qh, Kg, cd11,
                             preferred_element_type=f32)
    qk = qk * f32(1.0 / np.sqrt(HID // HEADS))

    A_i = a_ref[0]
    offs = offs_ref[:, :]
    off0 = CUTOFF / (ECH - 1)
    coeff = f32(-0.5 / (off0 * off0))
    sdb1 = sdb1_ref[:, :]
    tib1 = tib1_ref[:, :]
    for p in range(N_CH):
        dcol = Rkq[:, p:p + 1]
        dd = dcol - offs
        drep = jnp.exp(coeff * dd * dd)
        G = jax.lax.dot_general(drep, wg_ref[:, :], cd11,
                                preferred_element_type=f32)
        sdh = jnp.maximum(G[:, :HID] + sdb1, 0.0)
        sdp = jax.lax.dot_general(sdw2_ref[:, :], sdh, cd11,
                                  preferred_element_type=f32)
        sdp = sdp + sdb2_ref[:, :]
        tih = jnp.maximum(G[:, HID:] + A_i[p:p + 1, :] + Bg + tib1, 0.0)
        fp = jax.lax.dot_general(tiw2_ref[:, :], tih, cd11,
                                 preferred_element_type=f32)
        f_sc[p:p + 1, :] = fp + tib2_ref[:, :]
        for h in range(HEADS):
            r = h * N_CH + p
            l_sc[r:r + 1, :] = qk[r:r + 1, :] + sdp[h:h + 1, :]

    L = l_sc[:, :]
    ex = jnp.exp(L - jnp.max(L, axis=1, keepdims=True))
    den = jax.lax.dot_general(ex, bs_ref[:, :], cd10,
                              preferred_element_type=f32)
    aw = ex / den
    ami4 = jnp.concatenate([am_i_col] * HEADS, axis=0)
    aw = aw * (ami4 * am_j_row)
    asum = jax.lax.dot_general(aw, bs_ref[:, :], cd10,
                               preferred_element_type=f32)
    awn = aw / (asum + 1e-7)

    amL = (ami4 * L) * am_j_row
    rr = jax.lax.dot_general(sp_ref[:, :], amL, cd10,
                             preferred_element_type=f32)
    rk = jax.lax.dot_general(rr, Gc, cd10,
                             preferred_element_type=f32)
    amj_k = jax.lax.dot_general(am_j_row, Gc, cd10,
                                preferred_element_type=f32)
    denr = jnp.sum(am_i_row, axis=1, keepdims=True) * amj_k
    r = rk / denr
    er = jnp.exp(r - jnp.max(r, axis=1, keepdims=True))
    beta = er / jnp.sum(er, axis=1, keepdims=True)

    betaE = jax.lax.dot_general(beta, Gc, cd11,
                                preferred_element_type=f32)
    betaR = jax.lax.dot_general(sp_ref[:, :].T, betaE, cd10,
                                preferred_element_type=f32)
    awb = awn * betaR
    d = HID // HEADS
    upd = jnp.concatenate(
        [jax.lax.dot_general(awb[h * N_CH:(h + 1) * N_CH, :],
                             Vg[:, h * d:(h + 1) * d], cd10,
                             preferred_element_type=f32)
         for h in range(HEADS)], axis=1)
    sp = jnp.maximum(upd, 0.0) + jnp.log1p(jnp.exp(-jnp.abs(upd))) \
        - f32(np.log(2.0))
    ho = hln_ref[0] + jax.lax.dot_general(sp, wo_ref[:, :], cd11,
                                          preferred_element_type=f32)
    ho_ref[0] = ho * am_i_col

    meanAW = (awn[0:14, :] + awn[14:28, :] + awn[28:42, :] + awn[42:56, :]) \
        * f32(1.0 / HEADS)
    meanBeta = jnp.mean(beta, axis=0, keepdims=True)
    mbE = jax.lax.dot_general(meanBeta, Gc, cd11,
                              preferred_element_type=f32)
    Fw = f_sc[:, :] * meanAW * mbE
    Rpq = Rkq.T + 1e-5
    dcols = []
    for c in range(3):
        diff_c = xi[:, c * N_CH:(c + 1) * N_CH].T - xj_cols[c].T
        dn = diff_c / Rpq
        dcols.append(jnp.sum(Fw * dn, axis=1, keepdims=True))
    delta = jnp.clip(jnp.concatenate(dcols, axis=1), -3.0, 3.0)
    xo_ref[0] = xin_ref[0] + delta


def kernel(res_H, res_X, res_S, batch, ligand_pos, ligand_feat, ligand_mask,
           edit_residue_num, residue_mask, W_Q, W_K, W_V, W_O, ln_g, ln_b,
           sD_w1, sD_b1, sD_w2, sD_b2, Ti_w1, Ti_b1, Ti_w2, Ti_b2):
    n = N_NODES
    f32 = jnp.float32

    h2 = res_H.reshape(n * N_CH, HID)
    i21 = jnp.arange(21, dtype=jnp.int32)[None, :]
    ram = jnp.asarray(_RAM)
    nblk = 4
    rows = n * N_CH // nblk
    sblk = n // nblk
    hln2, Q2, K2, V2, A2, B2, am = pl.pallas_call(
        _prologue_body,
        grid=(nblk,),
        in_specs=[
            pl.BlockSpec((rows, HID), lambda i: (i, 0)),
            pl.BlockSpec((sblk, 1), lambda i: (i, 0)),
            pl.BlockSpec((HID, HID), lambda i: (0, 0)),
            pl.BlockSpec((HID, HID), lambda i: (0, 0)),
            pl.BlockSpec((HID, HID), lambda i: (0, 0)),
            pl.BlockSpec((HID, HID), lambda i: (0, 0)),
            pl.BlockSpec((HID, HID), lambda i: (0, 0)),
            pl.BlockSpec((1, HID), lambda i: (0, 0)),
            pl.BlockSpec((1, HID), lambda i: (0, 0)),
            pl.BlockSpec((1, 21), lambda i: (0, 0)),
            pl.BlockSpec((21, N_CH), lambda i: (0, 0)),
        ],
        out_specs=[pl.BlockSpec((rows, HID), lambda i: (i, 0))] * 6
        + [pl.BlockSpec((sblk, N_CH), lambda i: (i, 0))],
        out_shape=[jax.ShapeDtypeStruct((n * N_CH, HID), f32)] * 6
        + [jax.ShapeDtypeStruct((n, N_CH), f32)],
    )(h2, res_S.astype(jnp.int32)[:, None], W_Q, W_K, W_V,
      Ti_w1[:, :HID], Ti_w1[:, HID:2 * HID], ln_g[None, :], ln_b[None, :],
      i21, ram)

    pos = res_X[:, 1, :]
    bigeye = jnp.asarray(np.eye(n, dtype=np.float32) * 1e30)
    coli = jnp.arange(n, dtype=f32)[None, :]
    nbrf = pl.pallas_call(
        _knn_body,
        in_specs=[
            pl.BlockSpec((n, 3), lambda: (0, 0)),
            pl.BlockSpec((3, n), lambda: (0, 0)),
            pl.BlockSpec((n, n), lambda: (0, 0)),
            pl.BlockSpec((1, n), lambda: (0, 0)),
        ],
        out_specs=pl.BlockSpec((n, KNN), lambda: (0, 0)),
        out_shape=jax.ShapeDtypeStruct((n, KNN), f32),
    )(pos, pos.T, bigeye, coli)
    nbr9 = jnp.concatenate(
        [nbrf.astype(jnp.int32), jnp.arange(n, dtype=jnp.int32)[:, None]],
        axis=1)

    kq_i = np.arange(KQ)
    gc = jnp.asarray((kq_i[:, None] // N_CH == np.arange(K9)[None, :])
                     .astype(np.float32))
    qoh = jnp.asarray((kq_i[:, None] % N_CH == np.arange(N_CH)[None, :])
                      .astype(np.float32))
    bs = jnp.asarray((kq_i[:, None] // N_CH == kq_i[None, :] // N_CH)
                     .astype(np.float32))
    r56 = np.arange(HEADS * N_CH)
    sp_c = jnp.asarray((r56[None, :] // N_CH == np.arange(HEADS)[:, None])
                       .astype(np.float32))
    qm = jnp.asarray((np.arange(HID)[None, :] // (HID // HEADS)
                      == r56[:, None] // N_CH).astype(np.float32))
    offs = jnp.linspace(0.0, CUTOFF, ECH, dtype=f32)[None, :]
    wg2 = jnp.concatenate([sD_w1, Ti_w1[:, 2 * HID:]], axis=0)

    K3 = K2.reshape(n, N_CH, HID)
    V3 = V2.reshape(n, N_CH, HID)
    B3 = B2.reshape(n, N_CH, HID)
    Q3 = Q2.reshape(n, N_CH, HID)
    A3 = A2.reshape(n, N_CH, HID)
    hln3 = hln2.reshape(n, N_CH, HID)
    x42 = res_X.transpose(0, 2, 1).reshape(n, 1, 3 * N_CH)
    am3 = am.reshape(n, 1, N_CH)

    full3 = lambda a, b, c: pl.BlockSpec((a, b, c), lambda i: (0, 0, 0))
    blk3 = lambda b, c: pl.BlockSpec((1, b, c), lambda i: (i, 0, 0))
    full2 = lambda a, b: pl.BlockSpec((a, b), lambda i: (0, 0))
    h_out, x_out = pl.pallas_call(
        _main_body,
        grid=(n,),
        in_specs=[
            pl.BlockSpec(memory_space=pltpu.SMEM),
            full3(n, N_CH, HID), full3(n, N_CH, HID), full3(n, N_CH, HID),
            full3(n, 1, 3 * N_CH), full3(n, 1, N_CH),
            blk3(N_CH, HID), blk3(N_CH, HID), blk3(N_CH, HID),
            blk3(N_CH, 3),
            full2(2 * HID, ECH), full2(1, HID), full2(HEADS, HID),
            full2(HEADS, 1), full2(1, HID), full2(1, HID), full2(1, 1),
            full2(HID, HID), full2(1, ECH), full2(KQ, K9),
            full2(KQ, N_CH), full2(KQ, KQ), full2(HEADS, HEADS * N_CH),
            full2(HEADS * N_CH, HID),
        ],
        out_specs=[blk3(N_CH, HID), blk3(N_CH, 3)],
        out_shape=[jax.ShapeDtypeStruct((n, N_CH, HID), f32),
                   jax.ShapeDtypeStruct((n, N_CH, 3), f32)],
        scratch_shapes=[
            pltpu.VMEM((KQ, HID), f32), pltpu.VMEM((KQ, HID), f32),
            pltpu.VMEM((KQ, HID), f32), pltpu.VMEM((K9, 3 * N_CH), f32),
            pltpu.VMEM((K9, N_CH), f32), pltpu.VMEM((N_CH, KQ), f32),
            pltpu.VMEM((HEADS * N_CH, KQ), f32),
        ],
    )(nbr9, K3, V3, B3, x42, am3, Q3, A3, hln3, res_X,
      wg2, sD_b1[None, :], sD_w2, sD_b2[:, None], Ti_b1[None, :],
      Ti_w2, Ti_b2[None, :], W_O, offs, gc, qoh, bs, sp_c, qm)

    x_out = jnp.where(residue_mask[:, None, None], x_out, res_X)
    return h_out, x_out


# speedup vs baseline: 2.1963x; 1.3167x over previous
import functools

import jax
import jax.numpy as jnp
import numpy as np
from jax.experimental import pallas as pl
from jax.experimental.pallas import tpu as pltpu

N_NODES = 256
N_CH = 14
HID = 128
ECH = 64
HEADS = 4
KNN = 8
CUTOFF = 10.0
K9 = KNN + 1
KQ = K9 * N_CH

_RAM = np.array([
    [1, 1, 1, 1, 1, 1, 1, 1, 1, 1, 1, 1, 1, 1],
    [1, 1, 1, 1, 1, 0, 0, 0, 0, 0, 0, 0, 0, 0],
    [1, 1, 1, 1, 1, 1, 1, 1, 1, 1, 1, 0, 0, 0],
    [1, 1, 1, 1, 1, 1, 1, 1, 0, 0, 0, 0, 0, 0],
    [1, 1, 1, 1, 1, 1, 1, 1, 0, 0, 0, 0, 0, 0],
    [1, 1, 1, 1, 1, 1, 0, 0, 0, 0, 0, 0, 0, 0],
    [1, 1, 1, 1, 1, 1, 1, 1, 1, 0, 0, 0, 0, 0],
    [1, 1, 1, 1, 1, 1, 1, 1, 1, 0, 0, 0, 0, 0],
    [1, 1, 1, 1, 0, 0, 0, 0, 0, 0, 0, 0, 0, 0],
    [1, 1, 1, 1, 1, 1, 1, 1, 1, 1, 0, 0, 0, 0],
    [1, 1, 1, 1, 1, 1, 1, 1, 0, 0, 0, 0, 0, 0],
    [1, 1, 1, 1, 1, 1, 1, 1, 0, 0, 0, 0, 0, 0],
    [1, 1, 1, 1, 1, 1, 1, 1, 1, 0, 0, 0, 0, 0],
    [1, 1, 1, 1, 1, 1, 1, 1, 0, 0, 0, 0, 0, 0],
    [1, 1, 1, 1, 1, 1, 1, 1, 1, 1, 1, 0, 0, 0],
    [1, 1, 1, 1, 1, 1, 1, 0, 0, 0, 0, 0, 0, 0],
    [1, 1, 1, 1, 1, 1, 0, 0, 0, 0, 0, 0, 0, 0],
    [1, 1, 1, 1, 1, 1, 1, 0, 0, 0, 0, 0, 0, 0],
    [1, 1, 1, 1, 1, 1, 1, 1, 1, 1, 1, 1, 1, 1],
    [1, 1, 1, 1, 1, 1, 1, 1, 1, 1, 1, 1, 0, 0],
    [1, 1, 1, 1, 1, 1, 1, 0, 0, 0, 0, 0, 0, 0]], dtype=np.float32)


def _prologue_body(h_ref, s_ref, wq_ref, wk_ref, wv_ref, ta_ref, tb_ref,
                   g_ref, b_ref, i21_ref, ram_ref,
                   hln_ref, q_ref, k_ref, v_ref, a_ref, bb_ref, am_ref):
    x = h_ref[:, :]
    mu = jnp.mean(x, axis=1, keepdims=True)
    var = jnp.mean((x - mu) ** 2, axis=1, keepdims=True)
    hln = (x - mu) / jnp.sqrt(var + 1e-5) * g_ref[:, :] + b_ref[:, :]
    hln_ref[:, :] = hln
    cd = (((1,), (1,)), ((), ()))
    q = jax.lax.dot_general(hln, wq_ref[:, :], cd,
                            preferred_element_type=jnp.float32)
    k = jax.lax.dot_general(hln, wk_ref[:, :], cd,
                            preferred_element_type=jnp.float32)
    v = jax.lax.dot_general(hln, wv_ref[:, :], cd,
                            preferred_element_type=jnp.float32)
    q_ref[:, :] = q
    k_ref[:, :] = k
    v_ref[:, :] = v
    a_ref[:, :] = jax.lax.dot_general(q, ta_ref[:, :], cd,
                                      preferred_element_type=jnp.float32)
    bb_ref[:, :] = jax.lax.dot_general(k, tb_ref[:, :], cd,
                                       preferred_element_type=jnp.float32)
    oh = (s_ref[:, :] == i21_ref[:, :]).astype(jnp.float32)
    am_ref[:, :] = jax.lax.dot_general(
        oh, ram_ref[:, :], (((1,), (0,)), ((), ())),
        preferred_element_type=jnp.float32)


def _knn_body(pc_ref, pt_ref, eye_ref, coli_ref, nbr_ref):
    big = jnp.float32(1e30)
    d2 = eye_ref[:, :]
    for c in range(3):
        diff = pc_ref[:, c:c + 1] - pt_ref[c:c + 1, :]
        d2 = d2 + diff * diff
    coli = coli_ref[:, :]
    for k in range(KNN):
        m = jnp.min(d2, axis=1, keepdims=True)
        idx = jnp.min(jnp.where(d2 <= m, coli, jnp.float32(N_NODES)),
                      axis=1, keepdims=True)
        nbr_ref[:, k:k + 1] = idx
        d2 = jnp.where(coli == idx, big, d2)


TN = 4
RQ = TN * KQ
R56 = HEADS * N_CH


def _main_body(nbr_ref, k_ref, v_ref, b_ref, x_ref, am_ref,
               q_ref, a_ref, hln_ref, xin_ref,
               wg_ref, sdb1_ref, sdw2_ref, sdb2_ref, tib1_ref, tiw2_ref,
               tib2_ref, wo_ref, offs_ref, gcb_ref, gc_ref, qoh_ref,
               ns_ref, bs_ref, rsel_ref, selh_ref, qm_ref,
               ho_ref, xo_ref,
               kg, vg, bg, xg, amg, xgo, amgo):
    i = pl.program_id(0)
    cd11 = (((1,), (1,)), ((), ()))
    cd10 = (((1,), (0,)), ((), ()))
    f32 = jnp.float32
    dg = functools.partial(jax.lax.dot_general,
                          preferred_element_type=f32)

    for t in range(TN):
        nd = i * TN + t
        for k in range(K9):
            j = nbr_ref[nd, k]
            r0 = (t * K9 + k) * N_CH
            kg[r0:r0 + N_CH, :] = k_ref[j]
            vg[r0:r0 + N_CH, :] = v_ref[j]
            bg[r0:r0 + N_CH, :] = b_ref[j]
            xg[t * K9 + k:t * K9 + k + 1, :] = x_ref[j]
            amg[t * K9 + k:t * K9 + k + 1, :] = am_ref[j]
        xgo[t:t + 1, :] = x_ref[nd]
        amgo[t:t + 1, :] = am_ref[nd]

    Vg = vg[:, :]
    Bg = bg[:, :]
    GcB = gcb_ref[:, :]
    Gc = gc_ref[:, :]
    QOH = qoh_ref[:, :]
    NS = ns_ref[:, :]
    RSEL = rsel_ref[:, :]
    SELH = selh_ref[:, :]
    qmP = qm_ref[:, :]

    XREP = dg(GcB, xg[:, :], cd10)
    AMrep = dg(GcB, amg[:, :], cd10)
    XIrep = dg(NS, xgo[:, :], cd10)
    am_j_col = jnp.sum(AMrep * QOH, axis=1, keepdims=True)

    xj_cols = []
    d2kq = jnp.zeros((RQ, N_CH), f32)
    for c in range(3):
        xj_c = jnp.sum(XREP[:, c * N_CH:(c + 1) * N_CH] * QOH,
                       axis=1, keepdims=True)
        xj_cols.append(xj_c)
        dc = xj_c - XIrep[:, c * N_CH:(c + 1) * N_CH]
        d2kq = d2kq + dc * dc
    Rkq = jnp.sqrt(d2kq + 1e-12)

    qks = []
    for t in range(TN):
        QhP = dg(RSEL, q_ref[t], cd10) * qmP
        qks.append(dg(QhP, kg[t * KQ:(t + 1) * KQ, :], cd11))
    QK = jnp.concatenate(qks, axis=1) * f32(1.0 / np.sqrt(HID // HEADS))

    offs = offs_ref[:, :]
    off0 = CUTOFF / (ECH - 1)
    coeff = f32(-0.5 / (off0 * off0))
    sdb1 = sdb1_ref[:, :]
    tib1 = tib1_ref[:, :]
    l_rows = []
    f_rows = []
    for p in range(N_CH):
        dcol = Rkq[:, p:p + 1]
        dd = dcol - offs
        drep = jnp.exp(coeff * dd * dd)
        G = dg(drep, wg_ref[:, :], cd11)
        sdh = jnp.maximum(G[:, :HID] + sdb1, 0.0)
        sdp = dg(sdw2_ref[:, :], sdh, cd11) + sdb2_ref[:, :]
        A_p = dg(NS, jnp.concatenate(
            [a_ref[t, p:p + 1, :] for t in range(TN)], axis=0), cd10)
        tih = jnp.maximum(G[:, HID:] + A_p + Bg + tib1, 0.0)
        fp = dg(tiw2_ref[:, :], tih, cd11) + tib2_ref[:, :]
        l_rows.append(QK[p * HEADS:(p + 1) * HEADS, :] + sdp)
        f_rows.append(fp)

    L = jnp.concatenate(l_rows, axis=0)
    F = jnp.concatenate(f_rows, axis=0)

    ex = jnp.exp(L - jnp.max(L, axis=1, keepdims=True))
    den = jnp.concatenate(
        [dg(ex[:, t * KQ:(t + 1) * KQ], bs_ref[:, :], cd10)
         for t in range(TN)], axis=1)
    amirep = dg(RSEL, amgo[:, :].T, cd10)
    M = jnp.concatenate(
        [amirep[:, t:t + 1] * am_j_col[t * KQ:(t + 1) * KQ, :].T
         for t in range(TN)], axis=1)
    aw = (ex / den) * M
    asum = jnp.concatenate(
        [dg(aw[:, t * KQ:(t + 1) * KQ], bs_ref[:, :], cd10)
         for t in range(TN)], axis=1)
    awn = aw / (asum + 1e-7)

    amL = L * M
    for t in range(TN):
        tb = slice(t * KQ, (t + 1) * KQ)
        am_j_row = am_j_col[tb, :].T
        rr = dg(SELH, amL[:, tb], cd10)
        rk = dg(rr, Gc, cd10)
        amj_k = dg(am_j_row, Gc, cd10)
        am_i_row = amgo[t:t + 1, :]
        denr = jnp.sum(am_i_row, axis=1, keepdims=True) * amj_k
        r = rk / denr
        er = jnp.exp(r - jnp.max(r, axis=1, keepdims=True))
        beta = er / jnp.sum(er, axis=1, keepdims=True)

        betaE = dg(beta, Gc, cd11)
        betaR = dg(SELH.T, betaE, cd10)
        awb = awn[:, tb] * betaR
        updf = dg(awb, Vg[tb, :], cd10) * qmP
        upd = dg(RSEL.T, updf, cd10)
        sp = jnp.maximum(upd, 0.0) + jnp.log1p(jnp.exp(-jnp.abs(upd))) \
            - f32(np.log(2.0))
        ho = hln_ref[t] + dg(sp, wo_ref[:, :], cd11)
        ho_ref[t] = ho * am_i_row.T

        meanAW = dg(RSEL.T, awn[:, tb], cd10) * f32(1.0 / HEADS)
        meanBeta = jnp.mean(beta, axis=0, keepdims=True)
        mbE = dg(meanBeta, Gc, cd11)
        Fw = F[:, tb] * meanAW * mbE
        Rpq = Rkq[tb, :].T + 1e-5
        xi = xgo[t:t + 1, :]
        dcols = []
        for c in range(3):
            diff_c = (xi[:, c * N_CH:(c + 1) * N_CH].T
                      - xj_cols[c][tb, :].T)
            dn = diff_c / Rpq
            dcols.append(jnp.sum(Fw * dn, axis=1, keepdims=True))
        delta = jnp.clip(jnp.concatenate(dcols, axis=1), -3.0, 3.0)
        xo_ref[t] = xin_ref[t] + delta


def kernel(res_H, res_X, res_S, batch, ligand_pos, ligand_feat, ligand_mask,
           edit_residue_num, residue_mask, W_Q, W_K, W_V, W_O, ln_g, ln_b,
           sD_w1, sD_b1, sD_w2, sD_b2, Ti_w1, Ti_b1, Ti_w2, Ti_b2):
    n = N_NODES
    f32 = jnp.float32

    h2 = res_H.reshape(n * N_CH, HID)
    i21 = jnp.arange(21, dtype=jnp.int32)[None, :]
    ram = jnp.asarray(_RAM)
    nblk = 4
    rows = n * N_CH // nblk
    sblk = n // nblk
    hln2, Q2, K2, V2, A2, B2, am = pl.pallas_call(
        _prologue_body,
        grid=(nblk,),
        in_specs=[
            pl.BlockSpec((rows, HID), lambda i: (i, 0)),
            pl.BlockSpec((sblk, 1), lambda i: (i, 0)),
            pl.BlockSpec((HID, HID), lambda i: (0, 0)),
            pl.BlockSpec((HID, HID), lambda i: (0, 0)),
            pl.BlockSpec((HID, HID), lambda i: (0, 0)),
            pl.BlockSpec((HID, HID), lambda i: (0, 0)),
            pl.BlockSpec((HID, HID), lambda i: (0, 0)),
            pl.BlockSpec((1, HID), lambda i: (0, 0)),
            pl.BlockSpec((1, HID), lambda i: (0, 0)),
            pl.BlockSpec((1, 21), lambda i: (0, 0)),
            pl.BlockSpec((21, N_CH), lambda i: (0, 0)),
        ],
        out_specs=[pl.BlockSpec((rows, HID), lambda i: (i, 0))] * 6
        + [pl.BlockSpec((sblk, N_CH), lambda i: (i, 0))],
        out_shape=[jax.ShapeDtypeStruct((n * N_CH, HID), f32)] * 6
        + [jax.ShapeDtypeStruct((n, N_CH), f32)],
    )(h2, res_S.astype(jnp.int32)[:, None], W_Q, W_K, W_V,
      Ti_w1[:, :HID], Ti_w1[:, HID:2 * HID], ln_g[None, :], ln_b[None, :],
      i21, ram)

    pos = res_X[:, 1, :]
    bigeye = jnp.asarray(np.eye(n, dtype=np.float32) * 1e30)
    coli = jnp.arange(n, dtype=f32)[None, :]
    nbrf = pl.pallas_call(
        _knn_body,
        in_specs=[
            pl.BlockSpec((n, 3), lambda: (0, 0)),
            pl.BlockSpec((3, n), lambda: (0, 0)),
            pl.BlockSpec((n, n), lambda: (0, 0)),
            pl.BlockSpec((1, n), lambda: (0, 0)),
        ],
        out_specs=pl.BlockSpec((n, KNN), lambda: (0, 0)),
        out_shape=jax.ShapeDtypeStruct((n, KNN), f32),
    )(pos, pos.T, bigeye, coli)
    nbr9 = jnp.concatenate(
        [nbrf.astype(jnp.int32), jnp.arange(n, dtype=jnp.int32)[:, None]],
        axis=1)

    kq_i = np.arange(KQ)
    rq_i = np.arange(RQ)
    gc = jnp.asarray((kq_i[:, None] // N_CH == np.arange(K9)[None, :])
                     .astype(np.float32))
    gcb = jnp.asarray((rq_i[:, None] // N_CH == np.arange(TN * K9)[None, :])
                      .astype(np.float32))
    qoh = jnp.asarray((rq_i[:, None] % N_CH == np.arange(N_CH)[None, :])
                      .astype(np.float32))
    ns_c = jnp.asarray((rq_i[:, None] // KQ == np.arange(TN)[None, :])
                       .astype(np.float32))
    bs = jnp.asarray((kq_i[:, None] // N_CH == kq_i[None, :] // N_CH)
                     .astype(np.float32))
    r56 = np.arange(R56)
    rsel = jnp.asarray((r56[:, None] // HEADS == np.arange(N_CH)[None, :])
                       .astype(np.float32))
    selh = jnp.asarray((r56[None, :] % HEADS == np.arange(HEADS)[:, None])
                       .astype(np.float32))
    qm = jnp.asarray((np.arange(HID)[None, :] // (HID // HEADS)
                      == r56[:, None] % HEADS).astype(np.float32))
    offs = jnp.linspace(0.0, CUTOFF, ECH, dtype=f32)[None, :]
    wg2 = jnp.concatenate([sD_w1, Ti_w1[:, 2 * HID:]], axis=0)

    K3 = K2.reshape(n, N_CH, HID)
    V3 = V2.reshape(n, N_CH, HID)
    B3 = B2.reshape(n, N_CH, HID)
    Q3 = Q2.reshape(n, N_CH, HID)
    A3 = A2.reshape(n, N_CH, HID)
    hln3 = hln2.reshape(n, N_CH, HID)
    x42 = res_X.transpose(0, 2, 1).reshape(n, 1, 3 * N_CH)
    am3 = am.reshape(n, 1, N_CH)

    full3 = lambda a, b, c: pl.BlockSpec((a, b, c), lambda i: (0, 0, 0))
    blk3 = lambda b, c: pl.BlockSpec((TN, b, c), lambda i: (i, 0, 0))
    full2 = lambda a, b: pl.BlockSpec((a, b), lambda i: (0, 0))
    h_out, x_out = pl.pallas_call(
        _main_body,
        grid=(n // TN,),
        in_specs=[
            pl.BlockSpec(memory_space=pltpu.SMEM),
            full3(n, N_CH, HID), full3(n, N_CH, HID), full3(n, N_CH, HID),
            full3(n, 1, 3 * N_CH), full3(n, 1, N_CH),
            blk3(N_CH, HID), blk3(N_CH, HID), blk3(N_CH, HID),
            blk3(N_CH, 3),
            full2(2 * HID, ECH), full2(1, HID), full2(HEADS, HID),
            full2(HEADS, 1), full2(1, HID), full2(1, HID), full2(1, 1),
            full2(HID, HID), full2(1, ECH), full2(RQ, TN * K9),
            full2(KQ, K9), full2(RQ, N_CH), full2(RQ, TN),
            full2(KQ, KQ), full2(R56, N_CH), full2(HEADS, R56),
            full2(R56, HID),
        ],
        out_specs=[blk3(N_CH, HID), blk3(N_CH, 3)],
        out_shape=[jax.ShapeDtypeStruct((n, N_CH, HID), f32),
                   jax.ShapeDtypeStruct((n, N_CH, 3), f32)],
        scratch_shapes=[
            pltpu.VMEM((RQ, HID), f32), pltpu.VMEM((RQ, HID), f32),
            pltpu.VMEM((RQ, HID), f32), pltpu.VMEM((TN * K9, 3 * N_CH), f32),
            pltpu.VMEM((TN * K9, N_CH), f32), pltpu.VMEM((TN, 3 * N_CH), f32),
            pltpu.VMEM((TN, N_CH), f32),
        ],
    )(nbr9, K3, V3, B3, x42, am3, Q3, A3, hln3, res_X,
      wg2, sD_b1[None, :], sD_w2, sD_b2[:, None], Ti_b1[None, :],
      Ti_w2, Ti_b2[None, :], W_O, offs, gcb, gc, qoh, ns_c, bs, rsel,
      selh, qm)

    x_out = jnp.where(residue_mask[:, None, None], x_out, res_X)
    return h_out, x_out
